# baseline (speedup 1.0000x reference)
import functools

import jax
import jax.numpy as jnp
from jax import lax
from jax.experimental import pallas as pl
from jax.experimental.pallas import tpu as pltpu
from jax.experimental.pallas import tpu_sc as plsc

_B = 128
_U = 32768
_NC = 2
_NS = 16
_NW = _NC * _NS
_CW = _U // _NW
_RB = 32
_NRB = _B // _RB
_L = 16


@functools.partial(
    pl.kernel,
    mesh=plsc.VectorSubcoreMesh(core_axis_name="c", subcore_axis_name="s"),
    out_type=jax.ShapeDtypeStruct((_B, _U), jnp.float32),
    scratch_types=[
        pltpu.VMEM((_RB, _CW), jnp.float32),
        pltpu.VMEM((_RB, _CW), jnp.float32),
        pltpu.VMEM((_CW,), jnp.float32),
    ],
)
def _sc_mask(x_hbm, m_hbm, o_hbm, xv, ov, mv):
    wid = lax.axis_index("s") * _NC + lax.axis_index("c")
    c0 = wid * _CW
    pltpu.sync_copy(m_hbm.at[pl.ds(c0, _CW)], mv)
    for rb in range(_NRB):
        r0 = rb * _RB
        pltpu.sync_copy(x_hbm.at[pl.ds(r0, _RB), pl.ds(c0, _CW)], xv)

        def _row(r, carry):
            @plsc.parallel_loop(0, _CW // _L, 1, unroll=8)
            def _grp(g):
                sl = pl.ds(g * _L, _L)
                ov[r, sl] = jnp.where(mv[sl] != 0, xv[r, sl], jnp.float32(0))

            return carry

        lax.fori_loop(0, _RB, _row, 0)
        pltpu.sync_copy(ov, o_hbm.at[pl.ds(r0, _RB), pl.ds(c0, _CW)])


def kernel(inputs, mask):
    return _sc_mask(inputs, mask.astype(jnp.float32))


# speedup vs baseline: 1.0762x; 1.0762x over previous
import functools

import jax
import jax.numpy as jnp
from jax import lax
from jax.experimental import pallas as pl
from jax.experimental.pallas import tpu as pltpu
from jax.experimental.pallas import tpu_sc as plsc

_B = 128
_U = 32768
_TC_ROWS = 96
_SC_ROWS = _B - _TC_ROWS
_TC_BLK = _TC_ROWS // 2

_NC = 2
_NS = 16
_NW = _NC * _NS
_CW = _U // _NW
_L = 16


def _tc_body(x_ref, m_ref, o_ref):
    o_ref[...] = jnp.where(m_ref[...] != 0, x_ref[...], jnp.float32(0))


@functools.partial(
    pl.kernel,
    mesh=plsc.VectorSubcoreMesh(core_axis_name="c", subcore_axis_name="s"),
    out_type=jax.ShapeDtypeStruct((_SC_ROWS, _U), jnp.float32),
    scratch_types=[
        pltpu.VMEM((_SC_ROWS, _CW), jnp.float32),
        pltpu.VMEM((_SC_ROWS, _CW), jnp.float32),
        pltpu.VMEM((_CW,), jnp.float32),
    ],
)
def _sc_mask(x_hbm, m_hbm, o_hbm, xv, ov, mv):
    wid = lax.axis_index("s") * _NC + lax.axis_index("c")
    c0 = wid * _CW
    pltpu.sync_copy(m_hbm.at[pl.ds(c0, _CW)], mv)
    pltpu.sync_copy(x_hbm.at[pl.ds(_TC_ROWS, _SC_ROWS), pl.ds(c0, _CW)], xv)

    def _row(r, carry):
        @plsc.parallel_loop(0, _CW // _L, 1, unroll=8)
        def _grp(g):
            sl = pl.ds(g * _L, _L)
            ov[r, sl] = jnp.where(mv[sl] != 0, xv[r, sl], jnp.float32(0))

        return carry

    lax.fori_loop(0, _SC_ROWS, _row, 0)
    pltpu.sync_copy(ov, o_hbm.at[:, pl.ds(c0, _CW)])


def kernel(inputs, mask):
    b, u = inputs.shape
    m2 = mask.reshape(1, u).astype(jnp.int8)
    top = pl.pallas_call(
        _tc_body,
        grid=(2,),
        in_specs=[
            pl.BlockSpec((_TC_BLK, u), lambda i: (i, 0)),
            pl.BlockSpec((1, u), lambda i: (0, 0)),
        ],
        out_specs=pl.BlockSpec((_TC_BLK, u), lambda i: (i, 0)),
        out_shape=jax.ShapeDtypeStruct((_TC_ROWS, u), inputs.dtype),
        compiler_params=pltpu.CompilerParams(
            dimension_semantics=("parallel",),
        ),
    )(inputs, m2)
    bot = _sc_mask(inputs, mask.astype(jnp.float32))
    return jnp.concatenate([top, bot], axis=0)


# speedup vs baseline: 3.1084x; 2.8884x over previous
import jax
import jax.numpy as jnp
from jax.experimental import pallas as pl
from jax.experimental.pallas import tpu as pltpu

_ROWS = 64
_COLS = 16384


def _mask_body(x_ref, m_ref, o_ref):
    o_ref[...] = jnp.where(m_ref[...] != 0, x_ref[...], jnp.float32(0))


def kernel(inputs, mask):
    b, u = inputs.shape
    m2 = mask.reshape(1, u).astype(jnp.int8)
    grid = (b // _ROWS, u // _COLS)
    return pl.pallas_call(
        _mask_body,
        grid=grid,
        in_specs=[
            pl.BlockSpec((_ROWS, _COLS), lambda i, j: (i, j)),
            pl.BlockSpec((1, _COLS), lambda i, j: (0, j)),
        ],
        out_specs=pl.BlockSpec((_ROWS, _COLS), lambda i, j: (i, j)),
        out_shape=jax.ShapeDtypeStruct((b, u), inputs.dtype),
        compiler_params=pltpu.CompilerParams(
            dimension_semantics=("parallel", "arbitrary"),
        ),
    )(inputs, m2)


# speedup vs baseline: 3.5709x; 1.1488x over previous
import jax
import jax.numpy as jnp
from jax.experimental import pallas as pl
from jax.experimental.pallas import tpu as pltpu

_ROWS = 64


def _mask_body(x_ref, m_ref, o_ref):
    o_ref[...] = jnp.where(m_ref[...], x_ref[...], jnp.float32(0))


def kernel(inputs, mask):
    b, u = inputs.shape
    m2 = mask.reshape(1, u)
    grid = (b // _ROWS,)
    return pl.pallas_call(
        _mask_body,
        grid=grid,
        in_specs=[
            pl.BlockSpec((_ROWS, u), lambda i: (i, 0)),
            pl.BlockSpec((1, u), lambda i: (0, 0)),
        ],
        out_specs=pl.BlockSpec((_ROWS, u), lambda i: (i, 0)),
        out_shape=jax.ShapeDtypeStruct((b, u), inputs.dtype),
        compiler_params=pltpu.CompilerParams(
            dimension_semantics=("parallel",),
        ),
    )(inputs, m2)
